# Initial kernel scaffold; baseline (speedup 1.0000x reference)
import jax
import jax.numpy as jnp
from jax.experimental import pallas as pl


def kernel(x, edge_index, batch_graph, Ws1, Wn1, b1, Wfc1, bfc1, Ws2, Wn2, b2, Wfc2, bfc2, Ws3, Wn3, b3, Wfc3, bfc3, Wl1, bl1, Wl2, bl2, Wo, bo):
    raise NotImplementedError("write your pallas kernel here")


# speedup vs baseline: 5.0666x; 5.0666x over previous
import functools

import jax
import jax.numpy as jnp
from jax import lax
from jax.experimental import pallas as pl
from jax.experimental.pallas import tpu as pltpu
from jax.experimental.pallas import tpu_sc as plsc

N = 10000
E = 640000
ALPHA = 0.95
BETA = 1.0 - ALPHA

NS = 16
EPT = E // NS
K = 128
CHUNKS = EPT // K
TAIL = EPT - CHUNKS * K
ROWS_PT = 632
NP = ROWS_PT * NS


def _make_segsum(D2):
  mesh = plsc.VectorSubcoreMesh(core_axis_name="c", subcore_axis_name="s",
                                num_cores=2, num_subcores=NS)
  out_t = (jax.ShapeDtypeStruct((NP, D2), jnp.float32),
           jax.ShapeDtypeStruct((NP, D2), jnp.float32))

  def body(hlo, hhi, src, dst, zeros, alo, ahi,
           sidx, didx, rows, sidx_t, didx_t, rows_t, acc, sem):
    cid = lax.axis_index("c")
    sid = lax.axis_index("s")
    r0 = sid * ROWS_PT
    pltpu.sync_copy(zeros.at[pl.ds(r0, ROWS_PT)], acc.at[pl.ds(r0, ROWS_PT)])
    plsc.subcore_barrier()

    base = sid * EPT

    def chunk(off, idx_s, idx_d, buf):
      sz = idx_s.shape[1]
      pltpu.sync_copy(src.at[pl.ds(off, sz)], idx_s.at[0])
      pltpu.sync_copy(dst.at[pl.ds(off, sz)], idx_d.at[0])

      @pl.when(cid == 0)
      def _():
        pltpu.async_copy(hlo.at[idx_s.at[0]], buf, sem).wait()

      @pl.when(cid == 1)
      def _():
        pltpu.async_copy(hhi.at[idx_s.at[0]], buf, sem).wait()

      pltpu.sync_copy(buf, acc.at[idx_d.at[0]], add=True)

    def loop_body(j, carry):
      chunk(pl.multiple_of(base + j * K, 8), sidx, didx, rows)
      return carry

    lax.fori_loop(0, CHUNKS, loop_body, 0)
    chunk(base + CHUNKS * K, sidx_t, didx_t, rows_t)

    plsc.subcore_barrier()

    @pl.when(cid == 0)
    def _():
      pltpu.sync_copy(acc.at[pl.ds(r0, ROWS_PT)], alo.at[pl.ds(r0, ROWS_PT)])

    @pl.when(cid == 1)
    def _():
      pltpu.sync_copy(acc.at[pl.ds(r0, ROWS_PT)], ahi.at[pl.ds(r0, ROWS_PT)])

  return pl.kernel(
      body,
      out_type=out_t,
      mesh=mesh,
      compiler_params=pltpu.CompilerParams(use_tc_tiling_on_sc=False),
      scratch_types=[
          pltpu.VMEM((1, K), jnp.int32),
          pltpu.VMEM((1, K), jnp.int32),
          pltpu.VMEM((K, D2), jnp.float32),
          pltpu.VMEM((1, TAIL), jnp.int32),
          pltpu.VMEM((1, TAIL), jnp.int32),
          pltpu.VMEM((TAIL, D2), jnp.float32),
          pltpu.VMEM_SHARED((NP, D2), jnp.float32),
          pltpu.SemaphoreType.DMA,
      ],
  )


@functools.lru_cache(maxsize=None)
def _get_segsum(d2):
  return _make_segsum(d2)


def _agg(h, src, dst):
  d2 = h.shape[1] // 2
  alo, ahi = _get_segsum(d2)(h[:, :d2], h[:, d2:], src, dst,
                             jnp.zeros((NP, d2), jnp.float32))
  return jnp.concatenate([alo[:N], ahi[:N]], axis=1)


def _lrelu(v):
  return jnp.where(v > 0, v, 0.01 * v)


BN = 1000


def _dense_call(body, in_dims, out_dims):
  n_act = 2
  in_specs = []
  for i, d in enumerate(in_dims):
    if i < n_act:
      in_specs.append(pl.BlockSpec((BN, d), lambda i: (i, 0)))
    else:
      in_specs.append(pl.BlockSpec(d, lambda i: (0,) * len(d)))
  out_specs = [pl.BlockSpec((BN, d), lambda i: (i, 0)) for d in out_dims]
  out_shape = [jax.ShapeDtypeStruct((N, d), jnp.float32) for d in out_dims]
  if len(out_dims) == 1:
    out_specs, out_shape = out_specs[0], out_shape[0]
  return pl.pallas_call(
      body,
      grid=(N // BN,),
      in_specs=in_specs,
      out_specs=out_specs,
      out_shape=out_shape,
  )


def _mm(a, b):
  return jnp.dot(a, b, preferred_element_type=jnp.float32)


def _stage1_body(x, agg, ws, wn, b, wfc, bfc, out):
  h = jnp.maximum(ALPHA * _mm(x[...], ws[...]) + BETA * _mm(agg[...], wn[...])
                  + b[...], 0.0)
  out[...] = _lrelu(_mm(h, wfc[...]) + bfc[...])


def _stage2_body(t1, agg, ws, wn, b, wfc, bfc, wn3, t2_out, z_out):
  h = jnp.maximum(ALPHA * _mm(t1[...], ws[...]) + BETA * _mm(agg[...], wn[...])
                  + b[...], 0.0)
  t2 = _lrelu(_mm(h, wfc[...]) + bfc[...])
  t2_out[...] = t2
  z_out[...] = _mm(t2, wn3[...])


def _stage3_body(t2, agg, ws, b, wfc, bfc, wl1, bl1, wl2, bl2, wo, bo, out):
  h = jnp.maximum(ALPHA * _mm(t2[...], ws[...]) + BETA * agg[...] + b[...], 0.0)
  t3 = _lrelu(_mm(h, wfc[...]) + bfc[...])
  l1 = _mm(t3, wl1[...]) + bl1[...]
  l2 = _mm(l1, wl2[...]) + bl2[...]
  out[...] = jax.nn.sigmoid(_mm(l2, wo[...]) + bo[...])


_STAGE1 = _dense_call(
    _stage1_body,
    [128, 128, (128, 128), (128, 128), (1, 128), (128, 192), (1, 192)],
    [192])
_STAGE2 = _dense_call(
    _stage2_body,
    [192, 192, (192, 288), (192, 288), (1, 288), (288, 360), (1, 360),
     (360, 288)],
    [360, 288])
_STAGE3 = _dense_call(
    _stage3_body,
    [360, 288, (360, 288), (1, 288), (288, 192), (1, 192), (192, 128),
     (1, 128), (128, 64), (1, 64), (64, 8), (1, 8)],
    [8])


def _pad2(w, r, c):
  return jnp.pad(w, ((0, r - w.shape[0]), (0, c - w.shape[1])))


def _pad1(b, c):
  return jnp.pad(b, (0, c - b.shape[0])).reshape(1, c)


def kernel(x, edge_index, batch_graph, Ws1, Wn1, b1, Wfc1, bfc1, Ws2, Wn2, b2,
           Wfc2, bfc2, Ws3, Wn3, b3, Wfc3, bfc3, Wl1, bl1, Wl2, bl2, Wo, bo):
  src = edge_index[0].astype(jnp.int32)
  dst = edge_index[1].astype(jnp.int32)

  agg1 = _agg(x, src, dst)
  t1 = _STAGE1(x, agg1, Ws1, Wn1, b1.reshape(1, 128),
               _pad2(Wfc1, 128, 192), _pad1(bfc1, 192))

  agg2 = _agg(t1, src, dst)
  t2, z2 = _STAGE2(t1, agg2, _pad2(Ws2, 192, 288), _pad2(Wn2, 192, 288),
                   _pad1(b2, 288), _pad2(Wfc2, 288, 360), _pad1(bfc2, 360),
                   _pad2(Wn3, 360, 288))

  agg3 = _agg(z2, src, dst)
  out8 = _STAGE3(t2, agg3, _pad2(Ws3, 360, 288), _pad1(b3, 288),
                 _pad2(Wfc3, 288, 192), _pad1(bfc3, 192),
                 _pad2(Wl1, 192, 128), _pad1(bl1, 128),
                 _pad2(Wl2, 128, 64), _pad1(bl2, 64),
                 _pad2(Wo, 64, 8), _pad1(bo, 8))
  return out8[:, :6]


# speedup vs baseline: 8.0082x; 1.5806x over previous
import functools

import jax
import jax.numpy as jnp
from jax import lax
from jax.experimental import pallas as pl
from jax.experimental.pallas import tpu as pltpu
from jax.experimental.pallas import tpu_sc as plsc

N = 10000
E = 640000
ALPHA = 0.95
BETA = 1.0 - ALPHA

NS = 16
EPT = E // NS
K = 128
CHUNKS = EPT // K
TAIL = EPT - CHUNKS * K
QCH = CHUNKS // 4
ROWS_PT = 632
NP = ROWS_PT * NS


def _make_segsum(D2, nway):
  mesh = plsc.VectorSubcoreMesh(core_axis_name="c", subcore_axis_name="s",
                                num_cores=2, num_subcores=NS)
  out_t = tuple(jax.ShapeDtypeStruct((NP, D2), jnp.float32)
                for _ in range(nway))
  npass = nway // 2

  def body(*refs):
    hs = refs[:nway]
    srcm, dstm, srct, dstt, zeros = refs[nway:nway + 5]
    outs = refs[nway + 5:2 * nway + 5]
    (sidx, didx, rows0, rows1, sidx_t, didx_t, acc,
     gsem0, gsem1, ssem0, ssem1) = refs[2 * nway + 5:]
    cid = lax.axis_index("c")
    sid = lax.axis_index("s")
    r0 = sid * ROWS_PT
    cb = sid * CHUNKS

    rows = (rows0, rows1)
    gsem = (gsem0, gsem1)
    ssem = (ssem0, ssem1)

    def fire_gather(h0, h1, s, b):
      @pl.when(cid == 0)
      def _():
        pltpu.async_copy(h0.at[sidx.at[b]], rows[s], gsem[s])

      @pl.when(cid == 1)
      def _():
        pltpu.async_copy(h1.at[sidx.at[b]], rows[s], gsem[s])

    def drain_gather(s, b):
      pltpu.make_async_copy(hs[0].at[sidx.at[b]], rows[s], gsem[s]).wait()

    def fire_scatter(s, b):
      pltpu.async_copy(rows[s], acc.at[didx.at[b]], ssem[s], add=True)

    def drain_scatter(s, b):
      pltpu.make_async_copy(rows[s], acc.at[didx.at[b]], ssem[s]).wait()

    for p in range(npass):
      h0, h1 = hs[p], hs[npass + p]
      pltpu.sync_copy(zeros.at[pl.ds(r0, ROWS_PT)], acc.at[pl.ds(r0, ROWS_PT)])
      plsc.subcore_barrier()

      for q in range(4):
        qb = cb + q * QCH
        pltpu.sync_copy(srcm.at[pl.ds(qb, QCH)], sidx)
        pltpu.sync_copy(dstm.at[pl.ds(qb, QCH)], didx)
        fire_gather(h0, h1, 0, 0)
        fire_gather(h0, h1, 1, 1)

        def pair(j2, carry):
          b0 = j2 * 2
          drain_gather(0, b0)
          fire_scatter(0, b0)
          drain_gather(1, b0 + 1)
          fire_scatter(1, b0 + 1)
          drain_scatter(0, b0)
          fire_gather(h0, h1, 0, b0 + 2)
          drain_scatter(1, b0 + 1)
          fire_gather(h0, h1, 1, b0 + 3)
          return carry

        lax.fori_loop(0, QCH // 2 - 1, pair, 0)
        bl = QCH - 2
        drain_gather(0, bl)
        fire_scatter(0, bl)
        drain_gather(1, bl + 1)
        fire_scatter(1, bl + 1)
        drain_scatter(0, bl)
        drain_scatter(1, bl + 1)

      rt = rows0.at[pl.ds(0, TAIL)]
      pltpu.sync_copy(srct.at[pl.ds(sid, 1)], sidx_t)
      pltpu.sync_copy(dstt.at[pl.ds(sid, 1)], didx_t)

      @pl.when(cid == 0)
      def _():
        pltpu.async_copy(h0.at[sidx_t.at[0]], rt, gsem0).wait()

      @pl.when(cid == 1)
      def _():
        pltpu.async_copy(h1.at[sidx_t.at[0]], rt, gsem0).wait()

      pltpu.sync_copy(rt, acc.at[didx_t.at[0]], add=True)

      plsc.subcore_barrier()

      @pl.when(cid == 0)
      def _():
        pltpu.sync_copy(acc.at[pl.ds(r0, ROWS_PT)],
                        outs[p].at[pl.ds(r0, ROWS_PT)])

      @pl.when(cid == 1)
      def _():
        pltpu.sync_copy(acc.at[pl.ds(r0, ROWS_PT)],
                        outs[npass + p].at[pl.ds(r0, ROWS_PT)])

  return pl.kernel(
      body,
      out_type=out_t,
      mesh=mesh,
      compiler_params=pltpu.CompilerParams(use_tc_tiling_on_sc=False),
      scratch_types=[
          pltpu.VMEM((QCH, K), jnp.int32),
          pltpu.VMEM((QCH, K), jnp.int32),
          pltpu.VMEM((K, D2), jnp.float32),
          pltpu.VMEM((K, D2), jnp.float32),
          pltpu.VMEM((1, TAIL), jnp.int32),
          pltpu.VMEM((1, TAIL), jnp.int32),
          pltpu.VMEM_SHARED((NP, D2), jnp.float32),
          pltpu.SemaphoreType.DMA,
          pltpu.SemaphoreType.DMA,
          pltpu.SemaphoreType.DMA,
          pltpu.SemaphoreType.DMA,
      ],
  )


@functools.lru_cache(maxsize=None)
def _get_segsum(d2, nway):
  return _make_segsum(d2, nway)


def _agg(h, eidx, nway=2):
  d2 = h.shape[1] // nway
  parts = tuple(h[:, i * d2:(i + 1) * d2] for i in range(nway))
  outs = _get_segsum(d2, nway)(*parts, *eidx,
                               jnp.zeros((NP, d2), jnp.float32))
  return jnp.concatenate([o[:N] for o in outs], axis=1)


def _lrelu(v):
  return jnp.where(v > 0, v, 0.01 * v)


BN = 1000


def _dense_call(body, in_dims, out_dims):
  n_act = 2
  in_specs = []
  for i, d in enumerate(in_dims):
    if i < n_act:
      in_specs.append(pl.BlockSpec((BN, d), lambda i: (i, 0)))
    else:
      in_specs.append(pl.BlockSpec(d, lambda i: (0,) * len(d)))
  out_specs = [pl.BlockSpec((BN, d), lambda i: (i, 0)) for d in out_dims]
  out_shape = [jax.ShapeDtypeStruct((N, d), jnp.float32) for d in out_dims]
  if len(out_dims) == 1:
    out_specs, out_shape = out_specs[0], out_shape[0]
  return pl.pallas_call(
      body,
      grid=(N // BN,),
      in_specs=in_specs,
      out_specs=out_specs,
      out_shape=out_shape,
  )


def _mm(a, b):
  return jnp.dot(a, b, preferred_element_type=jnp.float32)


def _stage1_body(x, agg, ws, wn, b, wfc, bfc, out):
  h = jnp.maximum(ALPHA * _mm(x[...], ws[...]) + BETA * _mm(agg[...], wn[...])
                  + b[...], 0.0)
  out[...] = _lrelu(_mm(h, wfc[...]) + bfc[...])


def _stage2_body(t1, agg, ws, wn, b, wfc, bfc, wn3, t2_out, z_out):
  h = jnp.maximum(ALPHA * _mm(t1[...], ws[...]) + BETA * _mm(agg[...], wn[...])
                  + b[...], 0.0)
  t2 = _lrelu(_mm(h, wfc[...]) + bfc[...])
  t2_out[...] = t2
  z_out[...] = _mm(t2, wn3[...])


def _stage3_body(t2, agg, ws, b, wfc, bfc, wl1, bl1, wl2, bl2, wo, bo, out):
  h = jnp.maximum(ALPHA * _mm(t2[...], ws[...]) + BETA * agg[..., :288] + b[...], 0.0)
  t3 = _lrelu(_mm(h, wfc[...]) + bfc[...])
  l1 = _mm(t3, wl1[...]) + bl1[...]
  l2 = _mm(l1, wl2[...]) + bl2[...]
  out[...] = jax.nn.sigmoid(_mm(l2, wo[...]) + bo[...])


_STAGE1 = _dense_call(
    _stage1_body,
    [128, 128, (128, 128), (128, 128), (1, 128), (128, 192), (1, 192)],
    [192])
_STAGE2 = _dense_call(
    _stage2_body,
    [192, 192, (192, 288), (192, 288), (1, 288), (288, 360), (1, 360),
     (360, 320)],
    [360, 320])
_STAGE3 = _dense_call(
    _stage3_body,
    [360, 320, (360, 288), (1, 288), (288, 192), (1, 192), (192, 128),
     (1, 128), (128, 64), (1, 64), (64, 8), (1, 8)],
    [8])


def _pad2(w, r, c):
  return jnp.pad(w, ((0, r - w.shape[0]), (0, c - w.shape[1])))


def _pad1(b, c):
  return jnp.pad(b, (0, c - b.shape[0])).reshape(1, c)


def kernel(x, edge_index, batch_graph, Ws1, Wn1, b1, Wfc1, bfc1, Ws2, Wn2, b2,
           Wfc2, bfc2, Ws3, Wn3, b3, Wfc3, bfc3, Wl1, bl1, Wl2, bl2, Wo, bo):
  src = edge_index[0].astype(jnp.int32)
  dst = edge_index[1].astype(jnp.int32)
  nmain = NS * CHUNKS * K
  eidx = (src[:nmain].reshape(NS * CHUNKS, K),
          dst[:nmain].reshape(NS * CHUNKS, K),
          src[nmain:].reshape(NS, TAIL),
          dst[nmain:].reshape(NS, TAIL))

  agg1 = _agg(x, eidx)
  t1 = _STAGE1(x, agg1, Ws1, Wn1, b1.reshape(1, 128),
               _pad2(Wfc1, 128, 192), _pad1(bfc1, 192))

  agg2 = _agg(t1, eidx)
  t2, z2 = _STAGE2(t1, agg2, _pad2(Ws2, 192, 288), _pad2(Wn2, 192, 288),
                   _pad1(b2, 288), _pad2(Wfc2, 288, 360), _pad1(bfc2, 360),
                   _pad2(Wn3, 360, 320))

  agg3 = _agg(z2, eidx, nway=4)
  out8 = _STAGE3(t2, agg3, _pad2(Ws3, 360, 288), _pad1(b3, 288),
                 _pad2(Wfc3, 288, 192), _pad1(bfc3, 192),
                 _pad2(Wl1, 192, 128), _pad1(bl1, 128),
                 _pad2(Wl2, 128, 64), _pad1(bl2, 64),
                 _pad2(Wo, 64, 8), _pad1(bo, 8))
  return out8[:, :6]


# speedup vs baseline: 11.5335x; 1.4402x over previous
import functools

import jax
import jax.numpy as jnp
from jax import lax
from jax.experimental import pallas as pl
from jax.experimental.pallas import tpu as pltpu
from jax.experimental.pallas import tpu_sc as plsc

N = 10000
E = 640000
ALPHA = 0.95
BETA = 1.0 - ALPHA

NS = 16
EPT = E // NS
K = 80
CHUNKS = EPT // K
NBLK = 5
BLK = CHUNKS // NBLK
BLKQ = BLK // 4
ROWS_PT = 632
NP = ROWS_PT * NS


def _make_segsum(D2, nway):
  mesh = plsc.VectorSubcoreMesh(core_axis_name="c", subcore_axis_name="s",
                                num_cores=2, num_subcores=NS)
  out_t = tuple(jax.ShapeDtypeStruct((NP, D2), jnp.float32)
                for _ in range(nway))
  npass = nway // 2

  def body(*refs):
    hs = refs[:nway]
    srcm, dstm, zeros = refs[nway:nway + 3]
    outs = refs[nway + 3:2 * nway + 3]
    (sidx, didx, rows0, rows1, rows2, rows3, acc,
     gsem0, gsem1, gsem2, gsem3, ssem0, ssem1, ssem2, ssem3) = refs[2 * nway + 3:]
    cid = lax.axis_index("c")
    sid = lax.axis_index("s")
    r0 = sid * ROWS_PT
    cb = sid * CHUNKS

    rows = (rows0, rows1, rows2, rows3)
    gsem = (gsem0, gsem1, gsem2, gsem3)
    ssem = (ssem0, ssem1, ssem2, ssem3)

    def fire_gather(h0, h1, s, b):
      @pl.when(cid == 0)
      def _():
        pltpu.async_copy(h0.at[sidx.at[b]], rows[s], gsem[s])

      @pl.when(cid == 1)
      def _():
        pltpu.async_copy(h1.at[sidx.at[b]], rows[s], gsem[s])

    def drain_gather(s, b):
      pltpu.make_async_copy(hs[0].at[sidx.at[b]], rows[s], gsem[s]).wait()

    def fire_scatter(s, b):
      pltpu.async_copy(rows[s], acc.at[didx.at[b]], ssem[s], add=True)

    def drain_scatter(s, b):
      pltpu.make_async_copy(rows[s], acc.at[didx.at[b]], ssem[s]).wait()

    for p in range(npass):
      h0, h1 = hs[p], hs[npass + p]
      pltpu.sync_copy(zeros.at[pl.ds(r0, ROWS_PT)], acc.at[pl.ds(r0, ROWS_PT)])
      plsc.subcore_barrier()

      for blk in range(NBLK):
        qb = cb + blk * BLK
        pltpu.sync_copy(srcm.at[pl.ds(qb, BLK)], sidx)
        pltpu.sync_copy(dstm.at[pl.ds(qb, BLK)], didx)
        for i in range(4):
          fire_gather(h0, h1, i, i)
        drain_gather(0, 0)
        fire_scatter(0, 0)
        for i in (1, 2, 3):
          drain_gather(i, i)
          fire_scatter(i, i)
          drain_scatter(i - 1, i - 1)
          fire_gather(h0, h1, i - 1, i + 3)

        def quad(j, carry):
          b0 = j * 4
          for i in range(4):
            b = b0 + i
            drain_gather(i, b)
            fire_scatter(i, b)
            sp = (i - 1) % 4
            drain_scatter(sp, b - 1)
            fire_gather(h0, h1, sp, b + 3)
          return carry

        lax.fori_loop(1, BLKQ - 1, quad, 0)
        b0 = (BLKQ - 1) * 4
        for i in range(4):
          b = b0 + i
          drain_gather(i, b)
          fire_scatter(i, b)
          sp = (i - 1) % 4
          drain_scatter(sp, b - 1)
          if i == 0:
            fire_gather(h0, h1, sp, b + 3)
        drain_scatter(3, b0 + 3)

      plsc.subcore_barrier()

      @pl.when(cid == 0)
      def _():
        pltpu.sync_copy(acc.at[pl.ds(r0, ROWS_PT)],
                        outs[p].at[pl.ds(r0, ROWS_PT)])

      @pl.when(cid == 1)
      def _():
        pltpu.sync_copy(acc.at[pl.ds(r0, ROWS_PT)],
                        outs[npass + p].at[pl.ds(r0, ROWS_PT)])

  return pl.kernel(
      body,
      out_type=out_t,
      mesh=mesh,
      compiler_params=pltpu.CompilerParams(use_tc_tiling_on_sc=False),
      scratch_types=[
          pltpu.VMEM((BLK, K), jnp.int32),
          pltpu.VMEM((BLK, K), jnp.int32),
          pltpu.VMEM((K, D2), jnp.float32),
          pltpu.VMEM((K, D2), jnp.float32),
          pltpu.VMEM((K, D2), jnp.float32),
          pltpu.VMEM((K, D2), jnp.float32),
          pltpu.VMEM_SHARED((NP, D2), jnp.float32),
          pltpu.SemaphoreType.DMA,
          pltpu.SemaphoreType.DMA,
          pltpu.SemaphoreType.DMA,
          pltpu.SemaphoreType.DMA,
          pltpu.SemaphoreType.DMA,
          pltpu.SemaphoreType.DMA,
          pltpu.SemaphoreType.DMA,
          pltpu.SemaphoreType.DMA,
      ],
  )


@functools.lru_cache(maxsize=None)
def _get_segsum(d2, nway):
  return _make_segsum(d2, nway)


def _agg(h, eidx, nway=2):
  d2 = h.shape[1] // nway
  parts = tuple(h[:, i * d2:(i + 1) * d2] for i in range(nway))
  outs = _get_segsum(d2, nway)(*parts, *eidx,
                               jnp.zeros((NP, d2), jnp.float32))
  return jnp.concatenate([o[:N] for o in outs], axis=1)


def _lrelu(v):
  return jnp.where(v > 0, v, 0.01 * v)


BN = 1000


def _dense_call(body, in_dims, out_dims):
  n_act = 2
  in_specs = []
  for i, d in enumerate(in_dims):
    if i < n_act:
      in_specs.append(pl.BlockSpec((BN, d), lambda i: (i, 0)))
    else:
      in_specs.append(pl.BlockSpec(d, lambda i: (0,) * len(d)))
  out_specs = [pl.BlockSpec((BN, d), lambda i: (i, 0)) for d in out_dims]
  out_shape = [jax.ShapeDtypeStruct((N, d), jnp.float32) for d in out_dims]
  if len(out_dims) == 1:
    out_specs, out_shape = out_specs[0], out_shape[0]
  return pl.pallas_call(
      body,
      grid=(N // BN,),
      in_specs=in_specs,
      out_specs=out_specs,
      out_shape=out_shape,
  )


def _mm(a, b):
  return jnp.dot(a, b, preferred_element_type=jnp.float32)


def _stage1_body(x, agg, ws, wn, b, wfc, bfc, out):
  h = jnp.maximum(ALPHA * _mm(x[...], ws[...]) + BETA * _mm(agg[...], wn[...])
                  + b[...], 0.0)
  out[...] = _lrelu(_mm(h, wfc[...]) + bfc[...])


def _stage2_body(t1, agg, ws, wn, b, wfc, bfc, wn3, t2_out, z_out):
  h = jnp.maximum(ALPHA * _mm(t1[...], ws[...]) + BETA * _mm(agg[...], wn[...])
                  + b[...], 0.0)
  t2 = _lrelu(_mm(h, wfc[...]) + bfc[...])
  t2_out[...] = t2
  z_out[...] = _mm(t2, wn3[...])


def _stage3_body(t2, agg, ws, b, wfc, bfc, wl1, bl1, wl2, bl2, wo, bo, out):
  h = jnp.maximum(ALPHA * _mm(t2[...], ws[...]) + BETA * agg[..., :288] + b[...], 0.0)
  t3 = _lrelu(_mm(h, wfc[...]) + bfc[...])
  l1 = _mm(t3, wl1[...]) + bl1[...]
  l2 = _mm(l1, wl2[...]) + bl2[...]
  out[...] = jax.nn.sigmoid(_mm(l2, wo[...]) + bo[...])


_STAGE1 = _dense_call(
    _stage1_body,
    [128, 128, (128, 128), (128, 128), (1, 128), (128, 192), (1, 192)],
    [192])
_STAGE2 = _dense_call(
    _stage2_body,
    [192, 192, (192, 288), (192, 288), (1, 288), (288, 360), (1, 360),
     (360, 320)],
    [360, 320])
_STAGE3 = _dense_call(
    _stage3_body,
    [360, 320, (360, 288), (1, 288), (288, 192), (1, 192), (192, 128),
     (1, 128), (128, 64), (1, 64), (64, 8), (1, 8)],
    [8])


def _pad2(w, r, c):
  return jnp.pad(w, ((0, r - w.shape[0]), (0, c - w.shape[1])))


def _pad1(b, c):
  return jnp.pad(b, (0, c - b.shape[0])).reshape(1, c)


def kernel(x, edge_index, batch_graph, Ws1, Wn1, b1, Wfc1, bfc1, Ws2, Wn2, b2,
           Wfc2, bfc2, Ws3, Wn3, b3, Wfc3, bfc3, Wl1, bl1, Wl2, bl2, Wo, bo):
  src = edge_index[0].astype(jnp.int32)
  dst = edge_index[1].astype(jnp.int32)
  eidx = (src.reshape(NS * CHUNKS, K), dst.reshape(NS * CHUNKS, K))

  agg1 = _agg(x, eidx)
  t1 = _STAGE1(x, agg1, Ws1, Wn1, b1.reshape(1, 128),
               _pad2(Wfc1, 128, 192), _pad1(bfc1, 192))

  agg2 = _agg(t1, eidx)
  t2, z2 = _STAGE2(t1, agg2, _pad2(Ws2, 192, 288), _pad2(Wn2, 192, 288),
                   _pad1(b2, 288), _pad2(Wfc2, 288, 360), _pad1(bfc2, 360),
                   _pad2(Wn3, 360, 320))

  agg3 = _agg(z2, eidx, nway=4)
  out8 = _STAGE3(t2, agg3, _pad2(Ws3, 360, 288), _pad1(b3, 288),
                 _pad2(Wfc3, 288, 192), _pad1(bfc3, 192),
                 _pad2(Wl1, 192, 128), _pad1(bl1, 128),
                 _pad2(Wl2, 128, 64), _pad1(bl2, 64),
                 _pad2(Wo, 64, 8), _pad1(bo, 8))
  return out8[:, :6]


# speedup vs baseline: 17.0491x; 1.4782x over previous
import functools

import jax
import jax.numpy as jnp
from jax import lax
from jax.experimental import pallas as pl
from jax.experimental.pallas import tpu as pltpu
from jax.experimental.pallas import tpu_sc as plsc

N = 10000
E = 640000
ALPHA = 0.95
BETA = 1.0 - ALPHA

NS = 16
EPT = E // NS
K = 80
CHUNKS = EPT // K
NBLK = 5
BLK = CHUNKS // NBLK
BLKQ = BLK // 4
ROWS_PT = 632
NP = ROWS_PT * NS


def _make_segsum(D2, nway):
  mesh = plsc.VectorSubcoreMesh(core_axis_name="c", subcore_axis_name="s",
                                num_cores=2, num_subcores=NS)
  out_t = tuple(jax.ShapeDtypeStruct((NP, D2), jnp.bfloat16)
                for _ in range(nway))
  npass = nway // 2

  def body(*refs):
    hs = refs[:nway]
    srcm, dstm, zeros = refs[nway:nway + 3]
    outs = refs[nway + 3:2 * nway + 3]
    (sidx, didx, rows0, rows1, rows2, rows3, acc,
     gsem0, gsem1, gsem2, gsem3, ssem0, ssem1, ssem2, ssem3) = refs[2 * nway + 3:]
    cid = lax.axis_index("c")
    sid = lax.axis_index("s")
    r0 = sid * ROWS_PT
    cb = sid * CHUNKS

    rows = (rows0, rows1, rows2, rows3)
    gsem = (gsem0, gsem1, gsem2, gsem3)
    ssem = (ssem0, ssem1, ssem2, ssem3)

    def fire_gather(h0, h1, s, b):
      @pl.when(cid == 0)
      def _():
        pltpu.async_copy(h0.at[sidx.at[b]], rows[s], gsem[s])

      @pl.when(cid == 1)
      def _():
        pltpu.async_copy(h1.at[sidx.at[b]], rows[s], gsem[s])

    def drain_gather(s, b):
      pltpu.make_async_copy(hs[0].at[sidx.at[b]], rows[s], gsem[s]).wait()

    def fire_scatter(s, b):
      pltpu.async_copy(rows[s], acc.at[didx.at[b]], ssem[s], add=True)

    def drain_scatter(s, b):
      pltpu.make_async_copy(rows[s], acc.at[didx.at[b]], ssem[s]).wait()

    for p in range(npass):
      h0, h1 = hs[p], hs[npass + p]
      pltpu.sync_copy(zeros.at[pl.ds(r0, ROWS_PT)], acc.at[pl.ds(r0, ROWS_PT)])
      plsc.subcore_barrier()

      for blk in range(NBLK):
        qb = cb + blk * BLK
        pltpu.sync_copy(srcm.at[pl.ds(qb, BLK)], sidx)
        pltpu.sync_copy(dstm.at[pl.ds(qb, BLK)], didx)
        for i in range(4):
          fire_gather(h0, h1, i, i)
        drain_gather(0, 0)
        fire_scatter(0, 0)
        for i in (1, 2, 3):
          drain_gather(i, i)
          fire_scatter(i, i)
          drain_scatter(i - 1, i - 1)
          fire_gather(h0, h1, i - 1, i + 3)

        def quad(j, carry):
          b0 = j * 4
          for i in range(4):
            b = b0 + i
            drain_gather(i, b)
            fire_scatter(i, b)
            sp = (i - 1) % 4
            drain_scatter(sp, b - 1)
            fire_gather(h0, h1, sp, b + 3)
          return carry

        lax.fori_loop(1, BLKQ - 1, quad, 0)
        b0 = (BLKQ - 1) * 4
        for i in range(4):
          b = b0 + i
          drain_gather(i, b)
          fire_scatter(i, b)
          sp = (i - 1) % 4
          drain_scatter(sp, b - 1)
          if i == 0:
            fire_gather(h0, h1, sp, b + 3)
        drain_scatter(3, b0 + 3)

      plsc.subcore_barrier()

      @pl.when(cid == 0)
      def _():
        pltpu.sync_copy(acc.at[pl.ds(r0, ROWS_PT)],
                        outs[p].at[pl.ds(r0, ROWS_PT)])

      @pl.when(cid == 1)
      def _():
        pltpu.sync_copy(acc.at[pl.ds(r0, ROWS_PT)],
                        outs[npass + p].at[pl.ds(r0, ROWS_PT)])

  return pl.kernel(
      body,
      out_type=out_t,
      mesh=mesh,
      compiler_params=pltpu.CompilerParams(use_tc_tiling_on_sc=False),
      scratch_types=[
          pltpu.VMEM((BLK, K), jnp.int32),
          pltpu.VMEM((BLK, K), jnp.int32),
          pltpu.VMEM((K, D2), jnp.bfloat16),
          pltpu.VMEM((K, D2), jnp.bfloat16),
          pltpu.VMEM((K, D2), jnp.bfloat16),
          pltpu.VMEM((K, D2), jnp.bfloat16),
          pltpu.VMEM_SHARED((NP, D2), jnp.bfloat16),
          pltpu.SemaphoreType.DMA,
          pltpu.SemaphoreType.DMA,
          pltpu.SemaphoreType.DMA,
          pltpu.SemaphoreType.DMA,
          pltpu.SemaphoreType.DMA,
          pltpu.SemaphoreType.DMA,
          pltpu.SemaphoreType.DMA,
          pltpu.SemaphoreType.DMA,
      ],
  )


@functools.lru_cache(maxsize=None)
def _get_segsum(d2, nway):
  return _make_segsum(d2, nway)


def _agg(h, eidx, nway=2):
  d2 = h.shape[1] // nway
  parts = tuple(h[:, i * d2:(i + 1) * d2] for i in range(nway))
  outs = _get_segsum(d2, nway)(*parts, *eidx,
                               jnp.zeros((NP, d2), jnp.bfloat16))
  return jnp.concatenate([o[:N] for o in outs], axis=1)


def _lrelu(v):
  return jnp.where(v > 0, v, 0.01 * v)


BN = 1000


def _dense_call(body, in_dims, out_dims):
  n_act = 2
  in_specs = []
  for i, d in enumerate(in_dims):
    if i < n_act:
      in_specs.append(pl.BlockSpec((BN, d), lambda i: (i, 0)))
    else:
      in_specs.append(pl.BlockSpec(d, lambda i: (0,) * len(d)))
  out_specs = [pl.BlockSpec((BN, d), lambda i: (i, 0)) for d, _ in out_dims]
  out_shape = [jax.ShapeDtypeStruct((N, d), t) for d, t in out_dims]
  if len(out_dims) == 1:
    out_specs, out_shape = out_specs[0], out_shape[0]
  return pl.pallas_call(
      body,
      grid=(N // BN,),
      in_specs=in_specs,
      out_specs=out_specs,
      out_shape=out_shape,
  )


def _mm(a, b):
  return jnp.dot(a, b, preferred_element_type=jnp.float32)


def _stage1_body(x, agg, ws, wn, b, wfc, bfc, out, outb):
  a = agg[...].astype(jnp.float32)
  h = jnp.maximum(ALPHA * _mm(x[...], ws[...]) + BETA * _mm(a, wn[...])
                  + b[...], 0.0)
  t = _lrelu(_mm(h, wfc[...]) + bfc[...])
  out[...] = t
  outb[...] = t.astype(jnp.bfloat16)


def _stage2_body(t1, agg, ws, wn, b, wfc, bfc, wn3, t2_out, z_out):
  a = agg[...].astype(jnp.float32)
  h = jnp.maximum(ALPHA * _mm(t1[...], ws[...]) + BETA * _mm(a, wn[...])
                  + b[...], 0.0)
  t2 = _lrelu(_mm(h, wfc[...]) + bfc[...])
  t2_out[...] = t2
  z_out[...] = _mm(t2, wn3[...]).astype(jnp.bfloat16)


def _stage3_body(t2, agg, ws, b, wfc, bfc, wl1, bl1, wl2, bl2, wo, bo, out):
  a = agg[..., :288].astype(jnp.float32)
  h = jnp.maximum(ALPHA * _mm(t2[...], ws[...]) + BETA * a + b[...], 0.0)
  t3 = _lrelu(_mm(h, wfc[...]) + bfc[...])
  l1 = _mm(t3, wl1[...]) + bl1[...]
  l2 = _mm(l1, wl2[...]) + bl2[...]
  out[...] = jax.nn.sigmoid(_mm(l2, wo[...]) + bo[...])


_STAGE1 = _dense_call(
    _stage1_body,
    [128, 128, (128, 128), (128, 128), (1, 128), (128, 192), (1, 192)],
    [(192, jnp.float32), (192, jnp.bfloat16)])
_STAGE2 = _dense_call(
    _stage2_body,
    [192, 192, (192, 288), (192, 288), (1, 288), (288, 360), (1, 360),
     (360, 320)],
    [(360, jnp.float32), (320, jnp.bfloat16)])
_STAGE3 = _dense_call(
    _stage3_body,
    [360, 320, (360, 288), (1, 288), (288, 192), (1, 192), (192, 128),
     (1, 128), (128, 64), (1, 64), (64, 8), (1, 8)],
    [(8, jnp.float32)])


def _pad2(w, r, c):
  return jnp.pad(w, ((0, r - w.shape[0]), (0, c - w.shape[1])))


def _pad1(b, c):
  return jnp.pad(b, (0, c - b.shape[0])).reshape(1, c)


def kernel(x, edge_index, batch_graph, Ws1, Wn1, b1, Wfc1, bfc1, Ws2, Wn2, b2,
           Wfc2, bfc2, Ws3, Wn3, b3, Wfc3, bfc3, Wl1, bl1, Wl2, bl2, Wo, bo):
  src = edge_index[0].astype(jnp.int32)
  dst = edge_index[1].astype(jnp.int32)
  eidx = (src.reshape(NS * CHUNKS, K), dst.reshape(NS * CHUNKS, K))

  agg1 = _agg(x.astype(jnp.bfloat16), eidx)
  t1, t1b = _STAGE1(x, agg1, Ws1, Wn1, b1.reshape(1, 128),
                    _pad2(Wfc1, 128, 192), _pad1(bfc1, 192))

  agg2 = _agg(t1b, eidx)
  t2, z2 = _STAGE2(t1, agg2, _pad2(Ws2, 192, 288), _pad2(Wn2, 192, 288),
                   _pad1(b2, 288), _pad2(Wfc2, 288, 360), _pad1(bfc2, 360),
                   _pad2(Wn3, 360, 320))

  agg3 = _agg(z2, eidx)
  out8 = _STAGE3(t2, agg3, _pad2(Ws3, 360, 288), _pad1(b3, 288),
                 _pad2(Wfc3, 288, 192), _pad1(bfc3, 192),
                 _pad2(Wl1, 192, 128), _pad1(bl1, 128),
                 _pad2(Wl2, 128, 64), _pad1(bl2, 64),
                 _pad2(Wo, 64, 8), _pad1(bo, 8))
  return out8[:, :6]


# speedup vs baseline: 17.6565x; 1.0356x over previous
import functools

import jax
import jax.numpy as jnp
from jax import lax
from jax.experimental import pallas as pl
from jax.experimental.pallas import tpu as pltpu
from jax.experimental.pallas import tpu_sc as plsc

N = 10000
E = 640000
ALPHA = 0.95
BETA = 1.0 - ALPHA

NS = 16
EPT = E // NS
K = 80
CHUNKS = EPT // K
NBLK = 5
BLK = CHUNKS // NBLK
BLKQ = BLK // 4
ROWS_PT = 632
NP = ROWS_PT * NS


def _make_segsum(D2, nway):
  mesh = plsc.VectorSubcoreMesh(core_axis_name="c", subcore_axis_name="s",
                                num_cores=2, num_subcores=NS)
  out_t = tuple(jax.ShapeDtypeStruct((NP, D2), jnp.bfloat16)
                for _ in range(nway))
  npass = nway // 2

  def body(*refs):
    hs = refs[:nway]
    srcm, dstm, zeros = refs[nway:nway + 3]
    outs = refs[nway + 3:2 * nway + 3]
    (sidx, didx, rows0, rows1, rows2, rows3, acc,
     gsem0, gsem1, gsem2, gsem3, ssem0, ssem1, ssem2, ssem3) = refs[2 * nway + 3:]
    cid = lax.axis_index("c")
    sid = lax.axis_index("s")
    r0 = sid * ROWS_PT
    cb = sid * CHUNKS

    rows = (rows0, rows1, rows2, rows3)
    gsem = (gsem0, gsem1, gsem2, gsem3)
    ssem = (ssem0, ssem1, ssem2, ssem3)

    def fire_gather(h0, h1, s, b):
      @pl.when(cid == 0)
      def _():
        pltpu.async_copy(h0.at[sidx.at[b]], rows[s], gsem[s])

      @pl.when(cid == 1)
      def _():
        pltpu.async_copy(h1.at[sidx.at[b]], rows[s], gsem[s])

    def drain_gather(s, b):
      pltpu.make_async_copy(hs[0].at[sidx.at[b]], rows[s], gsem[s]).wait()

    def fire_scatter(s, b):
      pltpu.async_copy(rows[s], acc.at[didx.at[b]], ssem[s], add=True)

    def drain_scatter(s, b):
      pltpu.make_async_copy(rows[s], acc.at[didx.at[b]], ssem[s]).wait()

    for p in range(npass):
      h0, h1 = hs[p], hs[npass + p]
      pltpu.sync_copy(zeros.at[pl.ds(r0, ROWS_PT)], acc.at[pl.ds(r0, ROWS_PT)])
      plsc.subcore_barrier()

      for blk in range(NBLK):
        qb = cb + blk * BLK
        pltpu.sync_copy(srcm.at[pl.ds(qb, BLK)], sidx)
        pltpu.sync_copy(dstm.at[pl.ds(qb, BLK)], didx)
        for i in range(4):
          fire_gather(h0, h1, i, i)
        drain_gather(0, 0)
        fire_scatter(0, 0)
        for i in (1, 2, 3):
          drain_gather(i, i)
          fire_scatter(i, i)
          drain_scatter(i - 1, i - 1)
          fire_gather(h0, h1, i - 1, i + 3)

        def quad(j, carry):
          b0 = j * 4
          for i in range(4):
            b = b0 + i
            drain_gather(i, b)
            fire_scatter(i, b)
            sp = (i - 1) % 4
            drain_scatter(sp, b - 1)
            fire_gather(h0, h1, sp, b + 3)
          return carry

        lax.fori_loop(1, BLKQ - 1, quad, 0)
        b0 = (BLKQ - 1) * 4
        for i in range(4):
          b = b0 + i
          drain_gather(i, b)
          fire_scatter(i, b)
          sp = (i - 1) % 4
          drain_scatter(sp, b - 1)
          if i == 0:
            fire_gather(h0, h1, sp, b + 3)
        drain_scatter(3, b0 + 3)

      plsc.subcore_barrier()

      @pl.when(cid == 0)
      def _():
        pltpu.sync_copy(acc.at[pl.ds(r0, ROWS_PT)],
                        outs[p].at[pl.ds(r0, ROWS_PT)])

      @pl.when(cid == 1)
      def _():
        pltpu.sync_copy(acc.at[pl.ds(r0, ROWS_PT)],
                        outs[npass + p].at[pl.ds(r0, ROWS_PT)])

  return pl.kernel(
      body,
      out_type=out_t,
      mesh=mesh,
      compiler_params=pltpu.CompilerParams(use_tc_tiling_on_sc=False),
      scratch_types=[
          pltpu.VMEM((BLK, K), jnp.int32),
          pltpu.VMEM((BLK, K), jnp.int32),
          pltpu.VMEM((K, D2), jnp.bfloat16),
          pltpu.VMEM((K, D2), jnp.bfloat16),
          pltpu.VMEM((K, D2), jnp.bfloat16),
          pltpu.VMEM((K, D2), jnp.bfloat16),
          pltpu.VMEM_SHARED((NP, D2), jnp.bfloat16),
          pltpu.SemaphoreType.DMA,
          pltpu.SemaphoreType.DMA,
          pltpu.SemaphoreType.DMA,
          pltpu.SemaphoreType.DMA,
          pltpu.SemaphoreType.DMA,
          pltpu.SemaphoreType.DMA,
          pltpu.SemaphoreType.DMA,
          pltpu.SemaphoreType.DMA,
      ],
  )


@functools.lru_cache(maxsize=None)
def _get_segsum(d2, nway):
  return _make_segsum(d2, nway)


def _agg(hlo, hhi, eidx):
  return _get_segsum(hlo.shape[1], 2)(hlo, hhi, *eidx,
                                      jnp.zeros((NP, hlo.shape[1]),
                                                jnp.bfloat16))


def _lrelu(v):
  return jnp.where(v > 0, v, 0.01 * v)


BN = 1000


def _dense_call(body, in_dims, out_dims):
  n_act = 3
  in_specs = []
  for i, d in enumerate(in_dims):
    if i < n_act:
      in_specs.append(pl.BlockSpec((BN, d), lambda i: (i, 0)))
    else:
      in_specs.append(pl.BlockSpec(d, lambda i: (0,) * len(d)))
  out_specs = [pl.BlockSpec((BN, d), lambda i: (i, 0)) for d, _ in out_dims]
  out_shape = [jax.ShapeDtypeStruct((N, d), t) for d, t in out_dims]
  if len(out_dims) == 1:
    out_specs, out_shape = out_specs[0], out_shape[0]
  return pl.pallas_call(
      body,
      grid=(N // BN,),
      in_specs=in_specs,
      out_specs=out_specs,
      out_shape=out_shape,
  )


def _mm(a, b):
  return jnp.dot(a, b, preferred_element_type=jnp.float32)


def _stage1_body(x, alo, ahi, ws, wn, b, wfc, bfc, out, oblo, obhi):
  a = jnp.concatenate([alo[...], ahi[...]], axis=1).astype(jnp.float32)
  h = jnp.maximum(ALPHA * _mm(x[...], ws[...]) + BETA * _mm(a, wn[...])
                  + b[...], 0.0)
  t = _lrelu(_mm(h, wfc[...]) + bfc[...])
  out[...] = t
  tb = t.astype(jnp.bfloat16)
  oblo[...] = tb[:, :96]
  obhi[...] = tb[:, 96:]


def _stage2_body(t1, alo, ahi, ws, wn, b, wfc, bfc, wn3, t2_out, zlo, zhi):
  a = jnp.concatenate([alo[...], ahi[...]], axis=1).astype(jnp.float32)
  h = jnp.maximum(ALPHA * _mm(t1[...], ws[...]) + BETA * _mm(a, wn[...])
                  + b[...], 0.0)
  t2 = _lrelu(_mm(h, wfc[...]) + bfc[...])
  t2_out[...] = t2
  z = _mm(t2, wn3[...]).astype(jnp.bfloat16)
  zlo[...] = z[:, :160]
  zhi[...] = z[:, 160:]


def _stage3_body(t2, alo, ahi, ws, b, wfc, bfc, wl1, bl1, wl2, bl2, wo, bo,
                 out):
  a = jnp.concatenate([alo[...], ahi[..., :128]],
                      axis=1).astype(jnp.float32)
  h = jnp.maximum(ALPHA * _mm(t2[...], ws[...]) + BETA * a + b[...], 0.0)
  t3 = _lrelu(_mm(h, wfc[...]) + bfc[...])
  l1 = _mm(t3, wl1[...]) + bl1[...]
  l2 = _mm(l1, wl2[...]) + bl2[...]
  out[...] = jax.nn.sigmoid(_mm(l2, wo[...]) + bo[...])


_STAGE1 = _dense_call(
    _stage1_body,
    [128, 64, 64, (128, 128), (128, 128), (1, 128), (128, 192), (1, 192)],
    [(192, jnp.float32), (96, jnp.bfloat16), (96, jnp.bfloat16)])
_STAGE2 = _dense_call(
    _stage2_body,
    [192, 96, 96, (192, 288), (192, 288), (1, 288), (288, 360), (1, 360),
     (360, 320)],
    [(360, jnp.float32), (160, jnp.bfloat16), (160, jnp.bfloat16)])
_STAGE3 = _dense_call(
    _stage3_body,
    [360, 160, 128, (360, 288), (1, 288), (288, 192), (1, 192), (192, 128),
     (1, 128), (128, 64), (1, 64), (64, 8), (1, 8)],
    [(8, jnp.float32)])


def _pad2(w, r, c):
  return jnp.pad(w, ((0, r - w.shape[0]), (0, c - w.shape[1])))


def _pad1(b, c):
  return jnp.pad(b, (0, c - b.shape[0])).reshape(1, c)


def kernel(x, edge_index, batch_graph, Ws1, Wn1, b1, Wfc1, bfc1, Ws2, Wn2, b2,
           Wfc2, bfc2, Ws3, Wn3, b3, Wfc3, bfc3, Wl1, bl1, Wl2, bl2, Wo, bo):
  src = edge_index[0].astype(jnp.int32)
  dst = edge_index[1].astype(jnp.int32)
  eidx = (src.reshape(NS * CHUNKS, K), dst.reshape(NS * CHUNKS, K))

  xb = x.astype(jnp.bfloat16)
  a1lo, a1hi = _agg(xb[:, :64], xb[:, 64:], eidx)
  t1, t1blo, t1bhi = _STAGE1(x, a1lo, a1hi, Ws1, Wn1, b1.reshape(1, 128),
                             _pad2(Wfc1, 128, 192), _pad1(bfc1, 192))

  a2lo, a2hi = _agg(t1blo, t1bhi, eidx)
  t2, z2lo, z2hi = _STAGE2(t1, a2lo, a2hi, _pad2(Ws2, 192, 288),
                           _pad2(Wn2, 192, 288), _pad1(b2, 288),
                           _pad2(Wfc2, 288, 360), _pad1(bfc2, 360),
                           _pad2(Wn3, 360, 320))

  a3lo, a3hi = _agg(z2lo, z2hi, eidx)
  out8 = _STAGE3(t2, a3lo, a3hi, _pad2(Ws3, 360, 288), _pad1(b3, 288),
                 _pad2(Wfc3, 288, 192), _pad1(bfc3, 192),
                 _pad2(Wl1, 192, 128), _pad1(bl1, 128),
                 _pad2(Wl2, 128, 64), _pad1(bl2, 64),
                 _pad2(Wo, 64, 8), _pad1(bo, 8))
  return out8[:, :6]


# speedup vs baseline: 18.3295x; 1.0381x over previous
import functools

import jax
import jax.numpy as jnp
from jax import lax
from jax.experimental import pallas as pl
from jax.experimental.pallas import tpu as pltpu
from jax.experimental.pallas import tpu_sc as plsc

N = 10000
E = 640000
ALPHA = 0.95
BETA = 1.0 - ALPHA

NS = 16
EPT = E // NS
K = 80
CHUNKS = EPT // K
NBLK = 10
BLK = CHUNKS // NBLK
NSET = 5
BLKG = BLK // NSET
ROWS_PT = 632
NP = ROWS_PT * NS


def _make_segsum(D2, nway):
  mesh = plsc.VectorSubcoreMesh(core_axis_name="c", subcore_axis_name="s",
                                num_cores=2, num_subcores=NS)
  out_t = tuple(jax.ShapeDtypeStruct((NP, D2), jnp.bfloat16)
                for _ in range(nway))
  npass = nway // 2

  def body(*refs):
    hs = refs[:nway]
    srcm, dstm, zeros = refs[nway:nway + 3]
    outs = refs[nway + 3:2 * nway + 3]
    (sidx, didx, rows0, rows1, rows2, rows3, rows4, acc,
     gsem0, gsem1, gsem2, gsem3, gsem4,
     ssem0, ssem1, ssem2, ssem3, ssem4) = refs[2 * nway + 3:]
    cid = lax.axis_index("c")
    sid = lax.axis_index("s")
    r0 = sid * ROWS_PT
    cb = sid * CHUNKS

    rows = (rows0, rows1, rows2, rows3, rows4)
    gsem = (gsem0, gsem1, gsem2, gsem3, gsem4)
    ssem = (ssem0, ssem1, ssem2, ssem3, ssem4)

    def fire_gather(h0, h1, s, b):
      @pl.when(cid == 0)
      def _():
        pltpu.async_copy(h0.at[sidx.at[b]], rows[s], gsem[s])

      @pl.when(cid == 1)
      def _():
        pltpu.async_copy(h1.at[sidx.at[b]], rows[s], gsem[s])

    def drain_gather(s, b):
      pltpu.make_async_copy(hs[0].at[sidx.at[b]], rows[s], gsem[s]).wait()

    def fire_scatter(s, b):
      pltpu.async_copy(rows[s], acc.at[didx.at[b]], ssem[s], add=True)

    def drain_scatter(s, b):
      pltpu.make_async_copy(rows[s], acc.at[didx.at[b]], ssem[s]).wait()

    for p in range(npass):
      h0, h1 = hs[p], hs[npass + p]
      pltpu.sync_copy(zeros.at[pl.ds(r0, ROWS_PT)], acc.at[pl.ds(r0, ROWS_PT)])
      plsc.subcore_barrier()

      for blk in range(NBLK):
        qb = cb + blk * BLK
        pltpu.sync_copy(srcm.at[pl.ds(qb, BLK)], sidx)
        pltpu.sync_copy(dstm.at[pl.ds(qb, BLK)], didx)
        for i in range(NSET - 1):
          fire_gather(h0, h1, i, i)
        drain_gather(0, 0)
        fire_scatter(0, 0)
        fire_gather(h0, h1, NSET - 1, NSET - 1)
        for i in range(1, NSET):
          drain_gather(i, i)
          fire_scatter(i, i)
          drain_scatter(i - 1, i - 1)
          fire_gather(h0, h1, i - 1, i + NSET - 1)

        def grp(j, carry):
          b0 = j * NSET
          for i in range(NSET):
            b = b0 + i
            drain_gather(i, b)
            fire_scatter(i, b)
            sp = (i - 1) % NSET
            drain_scatter(sp, b - 1)
            fire_gather(h0, h1, sp, b + NSET - 1)
          return carry

        lax.fori_loop(1, BLKG - 1, grp, 0)
        b0 = (BLKG - 1) * NSET
        for i in range(NSET):
          b = b0 + i
          drain_gather(i, b)
          fire_scatter(i, b)
          sp = (i - 1) % NSET
          drain_scatter(sp, b - 1)
          if i == 0:
            fire_gather(h0, h1, sp, b + NSET - 1)
        drain_scatter(NSET - 1, b0 + NSET - 1)

      plsc.subcore_barrier()

      @pl.when(cid == 0)
      def _():
        pltpu.sync_copy(acc.at[pl.ds(r0, ROWS_PT)],
                        outs[p].at[pl.ds(r0, ROWS_PT)])

      @pl.when(cid == 1)
      def _():
        pltpu.sync_copy(acc.at[pl.ds(r0, ROWS_PT)],
                        outs[npass + p].at[pl.ds(r0, ROWS_PT)])

  return pl.kernel(
      body,
      out_type=out_t,
      mesh=mesh,
      compiler_params=pltpu.CompilerParams(use_tc_tiling_on_sc=False),
      scratch_types=[
          pltpu.VMEM((BLK, K), jnp.int32),
          pltpu.VMEM((BLK, K), jnp.int32),
          pltpu.VMEM((K, D2), jnp.bfloat16),
          pltpu.VMEM((K, D2), jnp.bfloat16),
          pltpu.VMEM((K, D2), jnp.bfloat16),
          pltpu.VMEM((K, D2), jnp.bfloat16),
          pltpu.VMEM((K, D2), jnp.bfloat16),
          pltpu.VMEM_SHARED((NP, D2), jnp.bfloat16),
      ] + [pltpu.SemaphoreType.DMA] * 10,
  )


@functools.lru_cache(maxsize=None)
def _get_segsum(d2, nway):
  return _make_segsum(d2, nway)


def _agg(hlo, hhi, eidx):
  return _get_segsum(hlo.shape[1], 2)(hlo, hhi, *eidx,
                                      jnp.zeros((NP, hlo.shape[1]),
                                                jnp.bfloat16))


def _lrelu(v):
  return jnp.where(v > 0, v, 0.01 * v)


BN = 1000


def _dense_call(body, in_dims, out_dims):
  n_act = 3
  in_specs = []
  for i, d in enumerate(in_dims):
    if i < n_act:
      in_specs.append(pl.BlockSpec((BN, d), lambda i: (i, 0)))
    else:
      in_specs.append(pl.BlockSpec(d, lambda i: (0,) * len(d)))
  out_specs = [pl.BlockSpec((BN, d), lambda i: (i, 0)) for d, _ in out_dims]
  out_shape = [jax.ShapeDtypeStruct((N, d), t) for d, t in out_dims]
  if len(out_dims) == 1:
    out_specs, out_shape = out_specs[0], out_shape[0]
  return pl.pallas_call(
      body,
      grid=(N // BN,),
      in_specs=in_specs,
      out_specs=out_specs,
      out_shape=out_shape,
  )


def _mm(a, b):
  return jnp.dot(a, b, preferred_element_type=jnp.float32)


def _stage1_body(x, alo, ahi, ws, wn, b, wfc, bfc, out, oblo, obhi):
  a = jnp.concatenate([alo[...], ahi[...]], axis=1).astype(jnp.float32)
  h = jnp.maximum(ALPHA * _mm(x[...], ws[...]) + BETA * _mm(a, wn[...])
                  + b[...], 0.0)
  t = _lrelu(_mm(h, wfc[...]) + bfc[...])
  out[...] = t
  tb = t.astype(jnp.bfloat16)
  oblo[...] = tb[:, :96]
  obhi[...] = tb[:, 96:]


def _stage2_body(t1, alo, ahi, ws, wn, b, wfc, bfc, wn3, t2_out, zlo, zhi):
  a = jnp.concatenate([alo[...], ahi[...]], axis=1).astype(jnp.float32)
  h = jnp.maximum(ALPHA * _mm(t1[...], ws[...]) + BETA * _mm(a, wn[...])
                  + b[...], 0.0)
  t2 = _lrelu(_mm(h, wfc[...]) + bfc[...])
  t2_out[...] = t2
  z = _mm(t2, wn3[...]).astype(jnp.bfloat16)
  zlo[...] = z[:, :160]
  zhi[...] = z[:, 160:]


def _stage3_body(t2, alo, ahi, ws, b, wfc, bfc, wl1, bl1, wl2, bl2, wo, bo,
                 out):
  a = jnp.concatenate([alo[...], ahi[..., :128]],
                      axis=1).astype(jnp.float32)
  h = jnp.maximum(ALPHA * _mm(t2[...], ws[...]) + BETA * a + b[...], 0.0)
  t3 = _lrelu(_mm(h, wfc[...]) + bfc[...])
  l1 = _mm(t3, wl1[...]) + bl1[...]
  l2 = _mm(l1, wl2[...]) + bl2[...]
  out[...] = jax.nn.sigmoid(_mm(l2, wo[...]) + bo[...])


_STAGE1 = _dense_call(
    _stage1_body,
    [128, 64, 64, (128, 128), (128, 128), (1, 128), (128, 192), (1, 192)],
    [(192, jnp.float32), (96, jnp.bfloat16), (96, jnp.bfloat16)])
_STAGE2 = _dense_call(
    _stage2_body,
    [192, 96, 96, (192, 288), (192, 288), (1, 288), (288, 360), (1, 360),
     (360, 320)],
    [(360, jnp.float32), (160, jnp.bfloat16), (160, jnp.bfloat16)])
_STAGE3 = _dense_call(
    _stage3_body,
    [360, 160, 128, (360, 288), (1, 288), (288, 192), (1, 192), (192, 128),
     (1, 128), (128, 64), (1, 64), (64, 8), (1, 8)],
    [(8, jnp.float32)])


def _pad2(w, r, c):
  return jnp.pad(w, ((0, r - w.shape[0]), (0, c - w.shape[1])))


def _pad1(b, c):
  return jnp.pad(b, (0, c - b.shape[0])).reshape(1, c)


def kernel(x, edge_index, batch_graph, Ws1, Wn1, b1, Wfc1, bfc1, Ws2, Wn2, b2,
           Wfc2, bfc2, Ws3, Wn3, b3, Wfc3, bfc3, Wl1, bl1, Wl2, bl2, Wo, bo):
  src = edge_index[0].astype(jnp.int32)
  dst = edge_index[1].astype(jnp.int32)
  eidx = (src.reshape(NS * CHUNKS, K), dst.reshape(NS * CHUNKS, K))

  xb = x.astype(jnp.bfloat16)
  a1lo, a1hi = _agg(xb[:, :64], xb[:, 64:], eidx)
  t1, t1blo, t1bhi = _STAGE1(x, a1lo, a1hi, Ws1, Wn1, b1.reshape(1, 128),
                             _pad2(Wfc1, 128, 192), _pad1(bfc1, 192))

  a2lo, a2hi = _agg(t1blo, t1bhi, eidx)
  t2, z2lo, z2hi = _STAGE2(t1, a2lo, a2hi, _pad2(Ws2, 192, 288),
                           _pad2(Wn2, 192, 288), _pad1(b2, 288),
                           _pad2(Wfc2, 288, 360), _pad1(bfc2, 360),
                           _pad2(Wn3, 360, 320))

  a3lo, a3hi = _agg(z2lo, z2hi, eidx)
  out8 = _STAGE3(t2, a3lo, a3hi, _pad2(Ws3, 360, 288), _pad1(b3, 288),
                 _pad2(Wfc3, 288, 192), _pad1(bfc3, 192),
                 _pad2(Wl1, 192, 128), _pad1(bl1, 128),
                 _pad2(Wl2, 128, 64), _pad1(bl2, 64),
                 _pad2(Wo, 64, 8), _pad1(bo, 8))
  return out8[:, :6]


# speedup vs baseline: 19.2158x; 1.0483x over previous
import functools

import jax
import jax.numpy as jnp
from jax import lax
from jax.experimental import pallas as pl
from jax.experimental.pallas import tpu as pltpu
from jax.experimental.pallas import tpu_sc as plsc

N = 10000
E = 640000
ALPHA = 0.95
BETA = 1.0 - ALPHA

NS = 16
EPT = E // NS
K = 80
CHUNKS = EPT // K
NBLK = 10
BLK = CHUNKS // NBLK
NSET = 5
BLKG = BLK // NSET
ROWS_PT = 632
NP = ROWS_PT * NS


def _make_segsum(D2, nway):
  mesh = plsc.VectorSubcoreMesh(core_axis_name="c", subcore_axis_name="s",
                                num_cores=2, num_subcores=NS)
  out_t = tuple(jax.ShapeDtypeStruct((NP, D2), jnp.bfloat16)
                for _ in range(nway))
  npass = nway // 2

  def body(*refs):
    hs = refs[:nway]
    srcm, dstm, zeros = refs[nway:nway + 3]
    outs = refs[nway + 3:2 * nway + 3]
    (sidx0, didx0, sidx1, didx1, rows0, rows1, rows2, rows3, rows4, acc,
     gsem0, gsem1, gsem2, gsem3, gsem4,
     ssem0, ssem1, ssem2, ssem3, ssem4, psem0, psem1) = refs[2 * nway + 3:]
    cid = lax.axis_index("c")
    sid = lax.axis_index("s")
    r0 = sid * ROWS_PT
    cb = sid * CHUNKS

    rows = (rows0, rows1, rows2, rows3, rows4)
    gsem = (gsem0, gsem1, gsem2, gsem3, gsem4)
    ssem = (ssem0, ssem1, ssem2, ssem3, ssem4)
    sidx = (sidx0, sidx1)
    didx = (didx0, didx1)
    psem = (psem0, psem1)

    def fire_preload(bsel, blk):
      off = cb + blk * BLK
      pltpu.async_copy(srcm.at[pl.ds(off, BLK)], sidx[bsel], psem[bsel])
      pltpu.async_copy(dstm.at[pl.ds(off, BLK)], didx[bsel], psem[bsel])

    def drain_preload(bsel, blk):
      off = cb + blk * BLK
      pltpu.make_async_copy(srcm.at[pl.ds(off, BLK)], sidx[bsel],
                            psem[bsel]).wait()
      pltpu.make_async_copy(dstm.at[pl.ds(off, BLK)], didx[bsel],
                            psem[bsel]).wait()

    def fire_gather(h0, h1, six, s, b):
      @pl.when(cid == 0)
      def _():
        pltpu.async_copy(h0.at[six.at[b]], rows[s], gsem[s])

      @pl.when(cid == 1)
      def _():
        pltpu.async_copy(h1.at[six.at[b]], rows[s], gsem[s])

    def drain_gather(six, s, b):
      pltpu.make_async_copy(hs[0].at[six.at[b]], rows[s], gsem[s]).wait()

    def fire_scatter(dix, s, b):
      pltpu.async_copy(rows[s], acc.at[dix.at[b]], ssem[s], add=True)

    def drain_scatter(dix, s, b):
      pltpu.make_async_copy(rows[s], acc.at[dix.at[b]], ssem[s]).wait()

    for p in range(npass):
      h0, h1 = hs[p], hs[npass + p]
      pltpu.sync_copy(zeros.at[pl.ds(r0, ROWS_PT)], acc.at[pl.ds(r0, ROWS_PT)])
      plsc.subcore_barrier()

      fire_preload(0, 0)
      for blk in range(NBLK):
        bsel = blk % 2
        six, dix = sidx[bsel], didx[bsel]
        drain_preload(bsel, blk)
        if blk + 1 < NBLK:
          fire_preload(1 - bsel, blk + 1)
        for i in range(NSET - 1):
          fire_gather(h0, h1, six, i, i)
        drain_gather(six, 0, 0)
        fire_scatter(dix, 0, 0)
        fire_gather(h0, h1, six, NSET - 1, NSET - 1)
        for i in range(1, NSET):
          drain_gather(six, i, i)
          fire_scatter(dix, i, i)
          drain_scatter(dix, i - 1, i - 1)
          fire_gather(h0, h1, six, i - 1, i + NSET - 1)

        def grp(j, carry):
          b0 = j * NSET
          for i in range(NSET):
            b = b0 + i
            drain_gather(six, i, b)
            fire_scatter(dix, i, b)
            sp = (i - 1) % NSET
            drain_scatter(dix, sp, b - 1)
            fire_gather(h0, h1, six, sp, b + NSET - 1)
          return carry

        lax.fori_loop(1, BLKG - 1, grp, 0)
        b0 = (BLKG - 1) * NSET
        for i in range(NSET):
          b = b0 + i
          drain_gather(six, i, b)
          fire_scatter(dix, i, b)
          sp = (i - 1) % NSET
          drain_scatter(dix, sp, b - 1)
          if i == 0:
            fire_gather(h0, h1, six, sp, b + NSET - 1)
        drain_scatter(dix, NSET - 1, b0 + NSET - 1)

      plsc.subcore_barrier()

      @pl.when(cid == 0)
      def _():
        pltpu.sync_copy(acc.at[pl.ds(r0, ROWS_PT)],
                        outs[p].at[pl.ds(r0, ROWS_PT)])

      @pl.when(cid == 1)
      def _():
        pltpu.sync_copy(acc.at[pl.ds(r0, ROWS_PT)],
                        outs[npass + p].at[pl.ds(r0, ROWS_PT)])

  return pl.kernel(
      body,
      out_type=out_t,
      mesh=mesh,
      compiler_params=pltpu.CompilerParams(use_tc_tiling_on_sc=False),
      scratch_types=[
          pltpu.VMEM((BLK, K), jnp.int32),
          pltpu.VMEM((BLK, K), jnp.int32),
          pltpu.VMEM((BLK, K), jnp.int32),
          pltpu.VMEM((BLK, K), jnp.int32),
          pltpu.VMEM((K, D2), jnp.bfloat16),
          pltpu.VMEM((K, D2), jnp.bfloat16),
          pltpu.VMEM((K, D2), jnp.bfloat16),
          pltpu.VMEM((K, D2), jnp.bfloat16),
          pltpu.VMEM((K, D2), jnp.bfloat16),
          pltpu.VMEM_SHARED((NP, D2), jnp.bfloat16),
      ] + [pltpu.SemaphoreType.DMA] * 12,
  )


@functools.lru_cache(maxsize=None)
def _get_segsum(d2, nway):
  return _make_segsum(d2, nway)


def _agg(hlo, hhi, eidx):
  return _get_segsum(hlo.shape[1], 2)(hlo, hhi, *eidx,
                                      jnp.zeros((NP, hlo.shape[1]),
                                                jnp.bfloat16))


def _lrelu(v):
  return jnp.where(v > 0, v, 0.01 * v)


BN = 1000


def _dense_call(body, in_dims, out_dims):
  n_act = 3
  in_specs = []
  for i, d in enumerate(in_dims):
    if i < n_act:
      in_specs.append(pl.BlockSpec((BN, d), lambda i: (i, 0)))
    else:
      in_specs.append(pl.BlockSpec(d, lambda i: (0,) * len(d)))
  out_specs = [pl.BlockSpec((BN, d), lambda i: (i, 0)) for d, _ in out_dims]
  out_shape = [jax.ShapeDtypeStruct((N, d), t) for d, t in out_dims]
  if len(out_dims) == 1:
    out_specs, out_shape = out_specs[0], out_shape[0]
  return pl.pallas_call(
      body,
      grid=(N // BN,),
      in_specs=in_specs,
      out_specs=out_specs,
      out_shape=out_shape,
  )


def _mm(a, b):
  return jnp.dot(a, b, preferred_element_type=jnp.float32)


def _stage1_body(x, alo, ahi, ws, wn, b, wfc, bfc, out, oblo, obhi):
  a = jnp.concatenate([alo[...], ahi[...]], axis=1).astype(jnp.float32)
  h = jnp.maximum(ALPHA * _mm(x[...], ws[...]) + BETA * _mm(a, wn[...])
                  + b[...], 0.0)
  t = _lrelu(_mm(h, wfc[...]) + bfc[...])
  out[...] = t
  tb = t.astype(jnp.bfloat16)
  oblo[...] = tb[:, :96]
  obhi[...] = tb[:, 96:]


def _stage2_body(t1, alo, ahi, ws, wn, b, wfc, bfc, wn3, t2_out, zlo, zhi):
  a = jnp.concatenate([alo[...], ahi[...]], axis=1).astype(jnp.float32)
  h = jnp.maximum(ALPHA * _mm(t1[...], ws[...]) + BETA * _mm(a, wn[...])
                  + b[...], 0.0)
  t2 = _lrelu(_mm(h, wfc[...]) + bfc[...])
  t2_out[...] = t2
  z = _mm(t2, wn3[...]).astype(jnp.bfloat16)
  zlo[...] = z[:, :160]
  zhi[...] = z[:, 160:]


def _stage3_body(t2, alo, ahi, ws, b, wfc, bfc, wl1, bl1, wl2, bl2, wo, bo,
                 out):
  a = jnp.concatenate([alo[...], ahi[..., :128]],
                      axis=1).astype(jnp.float32)
  h = jnp.maximum(ALPHA * _mm(t2[...], ws[...]) + BETA * a + b[...], 0.0)
  t3 = _lrelu(_mm(h, wfc[...]) + bfc[...])
  l1 = _mm(t3, wl1[...]) + bl1[...]
  l2 = _mm(l1, wl2[...]) + bl2[...]
  out[...] = jax.nn.sigmoid(_mm(l2, wo[...]) + bo[...])


_STAGE1 = _dense_call(
    _stage1_body,
    [128, 64, 64, (128, 128), (128, 128), (1, 128), (128, 192), (1, 192)],
    [(192, jnp.float32), (96, jnp.bfloat16), (96, jnp.bfloat16)])
_STAGE2 = _dense_call(
    _stage2_body,
    [192, 96, 96, (192, 288), (192, 288), (1, 288), (288, 360), (1, 360),
     (360, 320)],
    [(360, jnp.float32), (160, jnp.bfloat16), (160, jnp.bfloat16)])
_STAGE3 = _dense_call(
    _stage3_body,
    [360, 160, 128, (360, 288), (1, 288), (288, 192), (1, 192), (192, 128),
     (1, 128), (128, 64), (1, 64), (64, 8), (1, 8)],
    [(8, jnp.float32)])


def _pad2(w, r, c):
  return jnp.pad(w, ((0, r - w.shape[0]), (0, c - w.shape[1])))


def _pad1(b, c):
  return jnp.pad(b, (0, c - b.shape[0])).reshape(1, c)


def kernel(x, edge_index, batch_graph, Ws1, Wn1, b1, Wfc1, bfc1, Ws2, Wn2, b2,
           Wfc2, bfc2, Ws3, Wn3, b3, Wfc3, bfc3, Wl1, bl1, Wl2, bl2, Wo, bo):
  src = edge_index[0].astype(jnp.int32)
  dst = edge_index[1].astype(jnp.int32)
  eidx = (src.reshape(NS * CHUNKS, K), dst.reshape(NS * CHUNKS, K))

  xb = x.astype(jnp.bfloat16)
  a1lo, a1hi = _agg(xb[:, :64], xb[:, 64:], eidx)
  t1, t1blo, t1bhi = _STAGE1(x, a1lo, a1hi, Ws1, Wn1, b1.reshape(1, 128),
                             _pad2(Wfc1, 128, 192), _pad1(bfc1, 192))

  a2lo, a2hi = _agg(t1blo, t1bhi, eidx)
  t2, z2lo, z2hi = _STAGE2(t1, a2lo, a2hi, _pad2(Ws2, 192, 288),
                           _pad2(Wn2, 192, 288), _pad1(b2, 288),
                           _pad2(Wfc2, 288, 360), _pad1(bfc2, 360),
                           _pad2(Wn3, 360, 320))

  a3lo, a3hi = _agg(z2lo, z2hi, eidx)
  out8 = _STAGE3(t2, a3lo, a3hi, _pad2(Ws3, 360, 288), _pad1(b3, 288),
                 _pad2(Wfc3, 288, 192), _pad1(bfc3, 192),
                 _pad2(Wl1, 192, 128), _pad1(bl1, 128),
                 _pad2(Wl2, 128, 64), _pad1(bl2, 64),
                 _pad2(Wo, 64, 8), _pad1(bo, 8))
  return out8[:, :6]
